# baseline (speedup 1.0000x reference)
import functools

import jax
import jax.numpy as jnp
from jax import lax
from jax.experimental import pallas as pl
from jax.experimental.pallas import tpu as pltpu
from jax.experimental.pallas import tpu_sc as plsc

_N = 65536
_NNZ = 4194304
_NC = 2
_NS = 16
_NW = _NC * _NS
_SHARE = _NNZ // _NW
_W = 8192
_NWIN = _SHARE // _W
_L = 16
_AR = 128
_ACOL = _N // _AR


def _sc_segment_sum(indices, vals):
    mesh = plsc.VectorSubcoreMesh(core_axis_name="c", subcore_axis_name="s")

    @functools.partial(
        pl.kernel,
        mesh=mesh,
        out_type=jax.ShapeDtypeStruct((_NW, _AR, _ACOL), jnp.float32),
        compiler_params=pltpu.CompilerParams(needs_layout_passes=False),
        scratch_types=[
            pltpu.VMEM((_AR, _ACOL), jnp.float32),
            pltpu.VMEM((_W,), jnp.int32),
            pltpu.VMEM((_W,), jnp.int32),
            pltpu.VMEM((_W,), jnp.float32),
            pltpu.VMEM((_W,), jnp.float32),
            pltpu.SemaphoreType.DMA,
            pltpu.SemaphoreType.DMA,
            pltpu.SemaphoreType.DMA,
            pltpu.SemaphoreType.DMA,
        ],
    )
    def k(idx_hbm, vals_hbm, out_hbm, acc, idxw0, idxw1, valw0, valw1,
          si0, si1, sv0, sv1):
        c = lax.axis_index("c")
        s = lax.axis_index("s")
        wid = s * _NC + c
        base = wid * _SHARE

        bufs = ((idxw0, valw0, si0, sv0), (idxw1, valw1, si1, sv1))

        def start_win(g, b):
            iw, vw, si, sv = bufs[b]
            off = base + g * _W
            ci = pltpu.make_async_copy(idx_hbm.at[1, pl.ds(off, _W)], iw, si)
            cv = pltpu.make_async_copy(vals_hbm.at[pl.ds(off, _W)], vw, sv)
            ci.start()
            cv.start()
            return ci, cv

        handles = [start_win(0, 0), None]

        zeros = jnp.zeros((_L,), jnp.float32)

        def zrow(r, carry):
            for j in range(_ACOL // _L):
                acc[r, pl.ds(j * _L, _L)] = zeros
            return carry

        lax.fori_loop(0, _AR, zrow, 0)

        for g in range(_NWIN):
            b = g % 2
            nb = (g + 1) % 2
            if g + 1 < _NWIN:
                handles[nb] = start_win(g + 1, nb)
            hi, hv = handles[b]
            hi.wait()
            hv.wait()
            iw, vw = bufs[b][0], bufs[b][1]

            def ibody(i, icarry, iw=iw, vw=vw):
                sl = pl.ds(i * _L, _L)
                iv = iw[sl]
                plsc.addupdate_scatter(
                    acc, [iv >> 9, iv & (_ACOL - 1)], vw[sl])
                return icarry

            lax.fori_loop(0, _W // _L, ibody, 0, unroll=8)

        pltpu.sync_copy(acc, out_hbm.at[wid])

    return k(indices, vals)


def _combine_partials(partials):
    def body(p_ref, o1_ref, o2_ref):
        v = jnp.sum(p_ref[...], axis=0)
        o1_ref[...] = v
        o2_ref[...] = v

    return pl.pallas_call(
        body,
        out_shape=[
            jax.ShapeDtypeStruct((_N // 128, 128), jnp.float32),
            jax.ShapeDtypeStruct((_N // 128, 128), jnp.float32),
        ],
    )(partials)


def kernel(indices, values):
    if indices.dtype != jnp.int32:
        indices = indices.astype(jnp.int32)
    partials = _sc_segment_sum(indices, values)
    o1, o2 = _combine_partials(partials.reshape(_NW, _N // 128, 128))
    return (o1.reshape(_N), o2.reshape(_N))


# speedup vs baseline: 1.2559x; 1.2559x over previous
import functools

import jax
import jax.numpy as jnp
from jax import lax
from jax.experimental import pallas as pl
from jax.experimental.pallas import tpu as pltpu
from jax.experimental.pallas import tpu_sc as plsc

_N = 65536
_NNZ = 4194304
_NC = 2
_NS = 16
_NW = _NC * _NS
_SHARE = _NNZ // _NW
_W = 8192
_NWIN = _SHARE // _W
_L = 16
_SLABW = 8192
_NROUND = _N // _SLABW
_SLICE = _SLABW // _NS


def _sc_segment_sum(indices, vals):
    mesh = plsc.VectorSubcoreMesh(core_axis_name="c", subcore_axis_name="s")

    @functools.partial(
        pl.kernel,
        mesh=mesh,
        out_type=jax.ShapeDtypeStruct((_NC, _N), jnp.float32),
        compiler_params=pltpu.CompilerParams(needs_layout_passes=False),
        scratch_types=[
            pltpu.VMEM((_N,), jnp.float32),
            pltpu.VMEM((_W,), jnp.int32),
            pltpu.VMEM((_W,), jnp.int32),
            pltpu.VMEM((_W,), jnp.float32),
            pltpu.VMEM((_W,), jnp.float32),
            pltpu.VMEM((_NS, _SLICE), jnp.float32),
            pltpu.VMEM((_SLICE,), jnp.float32),
            pltpu.VMEM_SHARED((_NS, _SLABW), jnp.float32),
            pltpu.SemaphoreType.DMA,
            pltpu.SemaphoreType.DMA,
            pltpu.SemaphoreType.DMA,
            pltpu.SemaphoreType.DMA,
        ],
    )
    def k(idx_hbm, vals_hbm, out_hbm, acc, idxw0, idxw1, valw0, valw1,
          tmp2, red, slab, si0, si1, sv0, sv1):
        c = lax.axis_index("c")
        s = lax.axis_index("s")
        wid = s * _NC + c
        base = wid * _SHARE

        bufs = ((idxw0, valw0, si0, sv0), (idxw1, valw1, si1, sv1))

        def start_win(g, b):
            iw, vw, si, sv = bufs[b]
            off = base + g * _W
            ci = pltpu.make_async_copy(idx_hbm.at[1, pl.ds(off, _W)], iw, si)
            cv = pltpu.make_async_copy(vals_hbm.at[pl.ds(off, _W)], vw, sv)
            ci.start()
            cv.start()
            return ci, cv

        handles = [start_win(0, 0), None]

        zeros = jnp.zeros((_L,), jnp.float32)

        def zbody(i, carry):
            acc[pl.ds(i * _L, _L)] = zeros
            return carry

        lax.fori_loop(0, _N // _L, zbody, 0, unroll=16)

        for g in range(_NWIN):
            b = g % 2
            nb = (g + 1) % 2
            if g + 1 < _NWIN:
                handles[nb] = start_win(g + 1, nb)
            hi, hv = handles[b]
            hi.wait()
            hv.wait()
            iw, vw = bufs[b][0], bufs[b][1]

            def ibody(i, icarry, iw=iw, vw=vw):
                sl = pl.ds(i * _L, _L)
                plsc.addupdate_scatter(acc, [iw[sl]], vw[sl])
                return icarry

            lax.fori_loop(0, _W // _L, ibody, 0, unroll=8)

        for r in range(_NROUND):
            strip = r * _SLABW
            pltpu.sync_copy(acc.at[pl.ds(strip, _SLABW)], slab.at[s])
            plsc.subcore_barrier()

            off = s * _SLICE
            pltpu.sync_copy(slab.at[:, pl.ds(off, _SLICE)], tmp2)

            def abody(i, icarry):
                sl = pl.ds(i * _L, _L)
                v01 = tmp2[0, sl] + tmp2[1, sl]
                v23 = tmp2[2, sl] + tmp2[3, sl]
                v45 = tmp2[4, sl] + tmp2[5, sl]
                v67 = tmp2[6, sl] + tmp2[7, sl]
                v89 = tmp2[8, sl] + tmp2[9, sl]
                vab = tmp2[10, sl] + tmp2[11, sl]
                vcd = tmp2[12, sl] + tmp2[13, sl]
                vef = tmp2[14, sl] + tmp2[15, sl]
                red[sl] = (((v01 + v23) + (v45 + v67))
                           + ((v89 + vab) + (vcd + vef)))
                return icarry

            lax.fori_loop(0, _SLICE // _L, abody, 0, unroll=4)
            pltpu.sync_copy(red, out_hbm.at[c, pl.ds(strip + off, _SLICE)])
            plsc.subcore_barrier()

    return k(indices, vals)


def _combine_partials(partials):
    def body(p_ref, o1_ref, o2_ref):
        v = p_ref[0] + p_ref[1]
        o1_ref[...] = v
        o2_ref[...] = v

    return pl.pallas_call(
        body,
        out_shape=[
            jax.ShapeDtypeStruct((_N // 128, 128), jnp.float32),
            jax.ShapeDtypeStruct((_N // 128, 128), jnp.float32),
        ],
    )(partials)


def kernel(indices, values):
    if indices.dtype != jnp.int32:
        indices = indices.astype(jnp.int32)
    partials = _sc_segment_sum(indices, values)
    o1, o2 = _combine_partials(partials.reshape(_NC, _N // 128, 128))
    return (o1.reshape(_N), o2.reshape(_N))
